# baseline (speedup 1.0000x reference)
import functools

import jax
import jax.numpy as jnp
from jax.experimental import pallas as pl
from jax.experimental.pallas import tpu as pltpu


def _rowsum_kernel(m_ref, o_ref):
    o_ref[...] = jnp.sum(m_ref[...], axis=1, keepdims=True)


def _rowsum(m, bm=512):
    n = m.shape[0]
    return pl.pallas_call(
        _rowsum_kernel,
        grid=(n // bm,),
        in_specs=[pl.BlockSpec((bm, m.shape[1]), lambda i: (i, 0))],
        out_specs=pl.BlockSpec((bm, 1), lambda i: (i, 0)),
        out_shape=jax.ShapeDtypeStruct((n, 1), jnp.float32),
    )(m)


def _xw_kernel(x_ref, w_ref, deg_ref, o_ref):
    dinv = jax.lax.rsqrt(deg_ref[...])
    o_ref[...] = dinv * jnp.dot(x_ref[...], w_ref[...],
                                preferred_element_type=jnp.float32)


def _xw(x, w, deg, bm=1024):
    n, cin = x.shape
    cout = w.shape[1]
    return pl.pallas_call(
        _xw_kernel,
        grid=(n // bm,),
        in_specs=[
            pl.BlockSpec((bm, cin), lambda i: (i, 0)),
            pl.BlockSpec((cin, cout), lambda i: (0, 0)),
            pl.BlockSpec((bm, 1), lambda i: (i, 0)),
        ],
        out_specs=pl.BlockSpec((bm, cout), lambda i: (i, 0)),
        out_shape=jax.ShapeDtypeStruct((n, cout), jnp.float32),
    )(x, w, deg)


def _spmv_kernel(m_ref, u_ref, deg_ref, b_ref, o_ref, acc_ref, *, nk):
    k = pl.program_id(1)

    @pl.when(k == 0)
    def _():
        acc_ref[...] = jnp.zeros_like(acc_ref)

    acc_ref[...] += jnp.dot(m_ref[...], u_ref[...],
                            preferred_element_type=jnp.float32)

    @pl.when(k == nk - 1)
    def _():
        dinv = jax.lax.rsqrt(deg_ref[...])
        o_ref[...] = jnp.maximum(dinv * acc_ref[...] + b_ref[...], 0.0)


def _spmv_score_kernel(m_ref, u_ref, deg_ref, b_ref, p_ref, o_ref, s_ref,
                       acc_ref, *, nk):
    k = pl.program_id(1)

    @pl.when(k == 0)
    def _():
        acc_ref[...] = jnp.zeros_like(acc_ref)

    acc_ref[...] += jnp.dot(m_ref[...], u_ref[...],
                            preferred_element_type=jnp.float32)

    @pl.when(k == nk - 1)
    def _():
        dinv = jax.lax.rsqrt(deg_ref[...])
        y = jnp.maximum(dinv * acc_ref[...] + b_ref[...], 0.0)
        o_ref[...] = y
        p = p_ref[...]
        pn = p / jnp.sqrt(jnp.sum(p * p))
        s_ref[...] = jnp.dot(y, pn.T, preferred_element_type=jnp.float32)


def _conv_apply(m, u, deg, b, p=None, bm=512, bk=1024):
    n, _ = m.shape
    cout = u.shape[1]
    nk = m.shape[1] // bk
    b2 = b.reshape(1, cout)
    common = dict(
        grid=(n // bm, nk),
        out_shape=jax.ShapeDtypeStruct((n, cout), jnp.float32),
        scratch_shapes=[pltpu.VMEM((bm, cout), jnp.float32)],
        compiler_params=pltpu.CompilerParams(
            dimension_semantics=("parallel", "arbitrary")),
    )
    in_specs = [
        pl.BlockSpec((bm, bk), lambda i, k: (i, k)),
        pl.BlockSpec((bk, cout), lambda i, k: (k, 0)),
        pl.BlockSpec((bm, 1), lambda i, k: (i, 0)),
        pl.BlockSpec((1, cout), lambda i, k: (0, 0)),
    ]
    if p is None:
        return pl.pallas_call(
            functools.partial(_spmv_kernel, nk=nk),
            in_specs=in_specs,
            out_specs=pl.BlockSpec((bm, cout), lambda i, k: (i, 0)),
            **common,
        )(m, u, deg, b2)
    p2 = p.reshape(1, cout)
    y, s = pl.pallas_call(
        functools.partial(_spmv_score_kernel, nk=nk),
        in_specs=in_specs + [pl.BlockSpec((1, cout), lambda i, k: (0, 0))],
        out_specs=[
            pl.BlockSpec((bm, cout), lambda i, k: (i, 0)),
            pl.BlockSpec((bm, 1), lambda i, k: (i, 0)),
        ],
        out_shape=[
            jax.ShapeDtypeStruct((n, cout), jnp.float32),
            jax.ShapeDtypeStruct((n, 1), jnp.float32),
        ],
        scratch_shapes=[pltpu.VMEM((bm, cout), jnp.float32)],
        grid=(n // bm, nk),
        compiler_params=pltpu.CompilerParams(
            dimension_semantics=("parallel", "arbitrary")),
    )(m, u, deg, b2, p2)
    return y, s


def _sq_kernel(a_ref, c_ref, o_ref, acc_ref, *, nk, bm, bn, add_eye):
    i, j, k = pl.program_id(0), pl.program_id(1), pl.program_id(2)

    @pl.when(k == 0)
    def _():
        acc_ref[...] = jnp.zeros_like(acc_ref)

    acc_ref[...] += jnp.dot(a_ref[...], c_ref[...],
                            preferred_element_type=jnp.float32)

    @pl.when(k == nk - 1)
    def _():
        r = acc_ref[...]
        if add_eye:
            gi = i * bm + jax.lax.broadcasted_iota(jnp.int32, (bm, bn), 0)
            gj = j * bn + jax.lax.broadcasted_iota(jnp.int32, (bm, bn), 1)
            r = r + jnp.where(gi == gj, 1.0, 0.0)
        o_ref[...] = r


def _matmul(a, c, add_eye, bm=512, bn=512, bk=1024):
    m, kk = a.shape
    n = c.shape[1]
    nk = kk // bk
    return pl.pallas_call(
        functools.partial(_sq_kernel, nk=nk, bm=bm, bn=bn, add_eye=add_eye),
        grid=(m // bm, n // bn, nk),
        in_specs=[
            pl.BlockSpec((bm, bk), lambda i, j, k: (i, k)),
            pl.BlockSpec((bk, bn), lambda i, j, k: (k, j)),
        ],
        out_specs=pl.BlockSpec((bm, bn), lambda i, j, k: (i, j)),
        out_shape=jax.ShapeDtypeStruct((m, n), jnp.float32),
        scratch_shapes=[pltpu.VMEM((bm, bn), jnp.float32)],
        compiler_params=pltpu.CompilerParams(
            dimension_semantics=("parallel", "parallel", "arbitrary")),
    )(a, c)


def kernel(x, edge_index, W0, b0, W1, b1, W2, b2, W3, b3, W4, b4, p0, p1):
    n = x.shape[0]
    idx = jnp.arange(n)
    m0 = (jnp.zeros((n, n), jnp.float32)
          .at[edge_index[1], edge_index[0]].add(1.0)
          .at[idx, idx].add(1.0))
    deg0 = _rowsum(m0)

    u0 = _xw(x, W0, deg0)
    y0, s0 = _conv_apply(m0, u0, deg0, b0, p0)
    s0 = s0[:, 0]
    k0 = n // 2
    _, perm0 = jax.lax.top_k(s0, k0)
    x1 = y0[perm0] * jnp.tanh(s0[perm0])[:, None]
    m1 = _matmul(m0[perm0, :], m0[:, perm0], add_eye=True)
    deg1 = _rowsum(m1)

    u1 = _xw(x1, W1, deg1)
    y1, s1 = _conv_apply(m1, u1, deg1, b1, p1)
    s1 = s1[:, 0]
    k1 = k0 // 2
    _, perm1 = jax.lax.top_k(s1, k1)
    x2 = y1[perm1] * jnp.tanh(s1[perm1])[:, None]
    m2 = _matmul(m1[perm1, :], m1[:, perm1], add_eye=True, bk=512)
    deg2 = _rowsum(m2)

    u2 = _xw(x2, W2, deg2)
    y2 = _conv_apply(m2, u2, deg2, b2, bk=512)

    in3 = y1.at[perm1].add(y2)
    u3 = _xw(in3, W3, deg1)
    x3 = _conv_apply(m1, u3, deg1, b3)

    in4 = y0.at[perm0].add(x3)
    u4 = _xw(in4, W4, deg0)
    x4 = _conv_apply(m0, u4, deg0, b4)
    return x4


# speedup vs baseline: 1.0152x; 1.0152x over previous
import functools

import jax
import jax.numpy as jnp
from jax.experimental import pallas as pl
from jax.experimental.pallas import tpu as pltpu


def _rowsum_kernel(m_ref, o_ref):
    o_ref[...] = jnp.sum(m_ref[...], axis=1, keepdims=True)


def _rowsum(m, bm=512):
    n = m.shape[0]
    return pl.pallas_call(
        _rowsum_kernel,
        grid=(n // bm,),
        in_specs=[pl.BlockSpec((bm, m.shape[1]), lambda i: (i, 0))],
        out_specs=pl.BlockSpec((bm, 1), lambda i: (i, 0)),
        out_shape=jax.ShapeDtypeStruct((n, 1), jnp.float32),
    )(m)


def _xw_kernel(x_ref, w_ref, deg_ref, o_ref):
    dinv = jax.lax.rsqrt(deg_ref[...])
    o_ref[...] = dinv * jnp.dot(x_ref[...], w_ref[...],
                                preferred_element_type=jnp.float32)


def _xw(x, w, deg, bm=1024):
    n, cin = x.shape
    bm = min(bm, n)
    cout = w.shape[1]
    return pl.pallas_call(
        _xw_kernel,
        grid=(n // bm,),
        in_specs=[
            pl.BlockSpec((bm, cin), lambda i: (i, 0)),
            pl.BlockSpec((cin, cout), lambda i: (0, 0)),
            pl.BlockSpec((bm, 1), lambda i: (i, 0)),
        ],
        out_specs=pl.BlockSpec((bm, cout), lambda i: (i, 0)),
        out_shape=jax.ShapeDtypeStruct((n, cout), jnp.float32),
    )(x, w, deg)


def _spmv_kernel(m_ref, u_ref, deg_ref, b_ref, init_ref, o_ref, acc_ref, *,
                 nk, has_init):
    k = pl.program_id(1)

    @pl.when(k == 0)
    def _():
        if has_init:
            acc_ref[...] = init_ref[...]
        else:
            acc_ref[...] = jnp.zeros_like(acc_ref)

    acc_ref[...] += jnp.dot(m_ref[...], u_ref[...],
                            preferred_element_type=jnp.float32)

    @pl.when(k == nk - 1)
    def _():
        dinv = jax.lax.rsqrt(deg_ref[...])
        o_ref[...] = jnp.maximum(dinv * acc_ref[...] + b_ref[...], 0.0)


def _spmv_score_kernel(m_ref, u_ref, deg_ref, b_ref, p_ref, o_ref, s_ref,
                       acc_ref, *, nk):
    k = pl.program_id(1)

    @pl.when(k == 0)
    def _():
        acc_ref[...] = jnp.zeros_like(acc_ref)

    acc_ref[...] += jnp.dot(m_ref[...], u_ref[...],
                            preferred_element_type=jnp.float32)

    @pl.when(k == nk - 1)
    def _():
        dinv = jax.lax.rsqrt(deg_ref[...])
        y = jnp.maximum(dinv * acc_ref[...] + b_ref[...], 0.0)
        o_ref[...] = y
        p = p_ref[...]
        pn = p / jnp.sqrt(jnp.sum(p * p))
        s_ref[...] = jnp.dot(y, pn.T, preferred_element_type=jnp.float32)


def _conv_apply(m, u, deg, b, p=None, init=None, bm=512, bk=1024):
    n = m.shape[0]
    cout = u.shape[1]
    nk = m.shape[1] // bk
    b2 = b.reshape(1, cout)
    if p is None:
        has_init = init is not None
        if init is None:
            init = jnp.zeros((1, cout), jnp.float32)
            init_spec = pl.BlockSpec((1, cout), lambda i, k: (0, 0))
        else:
            init_spec = pl.BlockSpec((bm, cout), lambda i, k: (i, 0))
        return pl.pallas_call(
            functools.partial(_spmv_kernel, nk=nk, has_init=has_init),
            grid=(n // bm, nk),
            in_specs=[
                pl.BlockSpec((bm, bk), lambda i, k: (i, k)),
                pl.BlockSpec((bk, cout), lambda i, k: (k, 0)),
                pl.BlockSpec((bm, 1), lambda i, k: (i, 0)),
                pl.BlockSpec((1, cout), lambda i, k: (0, 0)),
                init_spec,
            ],
            out_specs=pl.BlockSpec((bm, cout), lambda i, k: (i, 0)),
            out_shape=jax.ShapeDtypeStruct((n, cout), jnp.float32),
            scratch_shapes=[pltpu.VMEM((bm, cout), jnp.float32)],
            compiler_params=pltpu.CompilerParams(
                dimension_semantics=("parallel", "arbitrary")),
        )(m, u, deg, b2, init)
    p2 = p.reshape(1, cout)
    y, s = pl.pallas_call(
        functools.partial(_spmv_score_kernel, nk=nk),
        grid=(n // bm, nk),
        in_specs=[
            pl.BlockSpec((bm, bk), lambda i, k: (i, k)),
            pl.BlockSpec((bk, cout), lambda i, k: (k, 0)),
            pl.BlockSpec((bm, 1), lambda i, k: (i, 0)),
            pl.BlockSpec((1, cout), lambda i, k: (0, 0)),
            pl.BlockSpec((1, cout), lambda i, k: (0, 0)),
        ],
        out_specs=[
            pl.BlockSpec((bm, cout), lambda i, k: (i, 0)),
            pl.BlockSpec((bm, 1), lambda i, k: (i, 0)),
        ],
        out_shape=[
            jax.ShapeDtypeStruct((n, cout), jnp.float32),
            jax.ShapeDtypeStruct((n, 1), jnp.float32),
        ],
        scratch_shapes=[pltpu.VMEM((bm, cout), jnp.float32)],
        compiler_params=pltpu.CompilerParams(
            dimension_semantics=("parallel", "arbitrary")),
    )(m, u, deg, b2, p2)
    return y, s


def _sq_kernel(a_ref, c_ref, o_ref, acc_ref, *, nk, bm, bn, add_eye):
    i, j, k = pl.program_id(0), pl.program_id(1), pl.program_id(2)

    @pl.when(k == 0)
    def _():
        acc_ref[...] = jnp.zeros_like(acc_ref)

    acc_ref[...] += jnp.dot(a_ref[...], c_ref[...],
                            preferred_element_type=jnp.float32)

    @pl.when(k == nk - 1)
    def _():
        r = acc_ref[...]
        if add_eye:
            gi = i * bm + jax.lax.broadcasted_iota(jnp.int32, (bm, bn), 0)
            gj = j * bn + jax.lax.broadcasted_iota(jnp.int32, (bm, bn), 1)
            r = r + jnp.where(gi == gj, 1.0, 0.0)
        o_ref[...] = r


def _matmul(a, c, add_eye=False, bm=512, bn=512, bk=1024):
    m, kk = a.shape
    n = c.shape[1]
    bk = min(bk, kk)
    bn = min(bn, n)
    nk = kk // bk
    return pl.pallas_call(
        functools.partial(_sq_kernel, nk=nk, bm=bm, bn=bn, add_eye=add_eye),
        grid=(m // bm, n // bn, nk),
        in_specs=[
            pl.BlockSpec((bm, bk), lambda i, j, k: (i, k)),
            pl.BlockSpec((bk, bn), lambda i, j, k: (k, j)),
        ],
        out_specs=pl.BlockSpec((bm, bn), lambda i, j, k: (i, j)),
        out_shape=jax.ShapeDtypeStruct((m, n), jnp.float32),
        scratch_shapes=[pltpu.VMEM((bm, bn), jnp.float32)],
        compiler_params=pltpu.CompilerParams(
            dimension_semantics=("parallel", "parallel", "arbitrary")),
    )(a, c)


def kernel(x, edge_index, W0, b0, W1, b1, W2, b2, W3, b3, W4, b4, p0, p1):
    n = x.shape[0]
    bf16 = jnp.bfloat16
    idx = jnp.arange(n, dtype=edge_index.dtype)
    rows = jnp.concatenate([edge_index[1], idx])
    cols = jnp.concatenate([edge_index[0], idx])
    m0 = jnp.zeros((n, n), jnp.float32).at[rows, cols].add(1.0)
    deg0 = _rowsum(m0)

    u0 = _xw(x, W0, deg0)
    y0, s0 = _conv_apply(m0, u0, deg0, b0, p0)
    s0 = s0[:, 0]
    k0 = n // 2
    _, perm0 = jax.lax.top_k(s0, k0)
    x1 = y0[perm0] * jnp.tanh(s0[perm0])[:, None]
    m0q = m0[:, perm0]
    m1 = _matmul(m0[perm0, :].astype(bf16), m0q.astype(bf16), add_eye=True)
    deg1 = _rowsum(m1)

    u1 = _xw(x1, W1, deg1)
    y1, s1 = _conv_apply(m1, u1, deg1, b1, p1)
    s1 = s1[:, 0]
    k1 = k0 // 2
    _, perm1 = jax.lax.top_k(s1, k1)
    x2 = y1[perm1] * jnp.tanh(s1[perm1])[:, None]
    m1q = m1[:, perm1]
    m2 = _matmul(m1[perm1, :].astype(bf16), m1q.astype(bf16), add_eye=True)
    deg2 = _rowsum(m2)

    u2 = _xw(x2, W2, deg2)
    y2 = _conv_apply(m2, u2, deg2, b2, bk=512)

    u3b = _xw(y2, W3, deg1[perm1])
    init3 = _matmul(m1q, u3b)
    u3a = _xw(y1, W3, deg1)
    x3 = _conv_apply(m1, u3a, deg1, b3, init=init3)

    u4b = _xw(x3, W4, deg0[perm0])
    init4 = _matmul(m0q, u4b)
    u4a = _xw(y0, W4, deg0)
    x4 = _conv_apply(m0, u4a, deg0, b4, init=init4)
    return x4


# speedup vs baseline: 1.0318x; 1.0163x over previous
import functools

import jax
import jax.numpy as jnp
from jax.experimental import pallas as pl
from jax.experimental.pallas import tpu as pltpu

_BF = jnp.bfloat16


def _split_dot(m_bf, u_f32, acc_dtype=jnp.float32):
    uh = u_f32.astype(_BF)
    ul = (u_f32 - uh.astype(jnp.float32)).astype(_BF)
    return (jnp.dot(m_bf, uh, preferred_element_type=acc_dtype)
            + jnp.dot(m_bf, ul, preferred_element_type=acc_dtype))


def _rowsum_kernel(m_ref, o_ref):
    o_ref[...] = jnp.sum(m_ref[...].astype(jnp.float32), axis=1,
                         keepdims=True)


def _rowsum(m, bm=512):
    n = m.shape[0]
    return pl.pallas_call(
        _rowsum_kernel,
        grid=(n // bm,),
        in_specs=[pl.BlockSpec((bm, m.shape[1]), lambda i: (i, 0))],
        out_specs=pl.BlockSpec((bm, 1), lambda i: (i, 0)),
        out_shape=jax.ShapeDtypeStruct((n, 1), jnp.float32),
    )(m)


def _xw_kernel(x_ref, w_ref, deg_ref, o_ref):
    dinv = jax.lax.rsqrt(deg_ref[...])
    o_ref[...] = dinv * jnp.dot(x_ref[...], w_ref[...],
                                preferred_element_type=jnp.float32)


def _xw(x, w, deg, bm=1024):
    n, cin = x.shape
    bm = min(bm, n)
    cout = w.shape[1]
    return pl.pallas_call(
        _xw_kernel,
        grid=(n // bm,),
        in_specs=[
            pl.BlockSpec((bm, cin), lambda i: (i, 0)),
            pl.BlockSpec((cin, cout), lambda i: (0, 0)),
            pl.BlockSpec((bm, 1), lambda i: (i, 0)),
        ],
        out_specs=pl.BlockSpec((bm, cout), lambda i: (i, 0)),
        out_shape=jax.ShapeDtypeStruct((n, cout), jnp.float32),
    )(x, w, deg)


def _spmv_kernel(m_ref, u_ref, deg_ref, b_ref, init_ref, o_ref, acc_ref, *,
                 nk, has_init, split):
    k = pl.program_id(1)

    @pl.when(k == 0)
    def _():
        if has_init:
            acc_ref[...] = init_ref[...]
        else:
            acc_ref[...] = jnp.zeros_like(acc_ref)

    if split:
        acc_ref[...] += _split_dot(m_ref[...], u_ref[...])
    else:
        acc_ref[...] += jnp.dot(m_ref[...], u_ref[...],
                                preferred_element_type=jnp.float32)

    @pl.when(k == nk - 1)
    def _():
        dinv = jax.lax.rsqrt(deg_ref[...])
        o_ref[...] = jnp.maximum(dinv * acc_ref[...] + b_ref[...], 0.0)


def _spmv_score_kernel(m_ref, u_ref, deg_ref, b_ref, p_ref, o_ref, s_ref,
                       acc_ref, *, nk, split):
    k = pl.program_id(1)

    @pl.when(k == 0)
    def _():
        acc_ref[...] = jnp.zeros_like(acc_ref)

    if split:
        acc_ref[...] += _split_dot(m_ref[...], u_ref[...])
    else:
        acc_ref[...] += jnp.dot(m_ref[...], u_ref[...],
                                preferred_element_type=jnp.float32)

    @pl.when(k == nk - 1)
    def _():
        dinv = jax.lax.rsqrt(deg_ref[...])
        y = jnp.maximum(dinv * acc_ref[...] + b_ref[...], 0.0)
        o_ref[...] = y
        p = p_ref[...]
        pn = p / jnp.sqrt(jnp.sum(p * p))
        s_ref[...] = jnp.dot(y, pn.T, preferred_element_type=jnp.float32)


def _conv_apply(m, u, deg, b, p=None, init=None, bm=512, bk=1024):
    n = m.shape[0]
    cout = u.shape[1]
    nk = m.shape[1] // bk
    b2 = b.reshape(1, cout)
    split = m.dtype == _BF
    if p is None:
        has_init = init is not None
        if init is None:
            init = jnp.zeros((1, cout), jnp.float32)
            init_spec = pl.BlockSpec((1, cout), lambda i, k: (0, 0))
        else:
            init_spec = pl.BlockSpec((bm, cout), lambda i, k: (i, 0))
        return pl.pallas_call(
            functools.partial(_spmv_kernel, nk=nk, has_init=has_init,
                              split=split),
            grid=(n // bm, nk),
            in_specs=[
                pl.BlockSpec((bm, bk), lambda i, k: (i, k)),
                pl.BlockSpec((bk, cout), lambda i, k: (k, 0)),
                pl.BlockSpec((bm, 1), lambda i, k: (i, 0)),
                pl.BlockSpec((1, cout), lambda i, k: (0, 0)),
                init_spec,
            ],
            out_specs=pl.BlockSpec((bm, cout), lambda i, k: (i, 0)),
            out_shape=jax.ShapeDtypeStruct((n, cout), jnp.float32),
            scratch_shapes=[pltpu.VMEM((bm, cout), jnp.float32)],
            compiler_params=pltpu.CompilerParams(
                dimension_semantics=("parallel", "arbitrary")),
        )(m, u, deg, b2, init)
    p2 = p.reshape(1, cout)
    y, s = pl.pallas_call(
        functools.partial(_spmv_score_kernel, nk=nk, split=split),
        grid=(n // bm, nk),
        in_specs=[
            pl.BlockSpec((bm, bk), lambda i, k: (i, k)),
            pl.BlockSpec((bk, cout), lambda i, k: (k, 0)),
            pl.BlockSpec((bm, 1), lambda i, k: (i, 0)),
            pl.BlockSpec((1, cout), lambda i, k: (0, 0)),
            pl.BlockSpec((1, cout), lambda i, k: (0, 0)),
        ],
        out_specs=[
            pl.BlockSpec((bm, cout), lambda i, k: (i, 0)),
            pl.BlockSpec((bm, 1), lambda i, k: (i, 0)),
        ],
        out_shape=[
            jax.ShapeDtypeStruct((n, cout), jnp.float32),
            jax.ShapeDtypeStruct((n, 1), jnp.float32),
        ],
        scratch_shapes=[pltpu.VMEM((bm, cout), jnp.float32)],
        compiler_params=pltpu.CompilerParams(
            dimension_semantics=("parallel", "arbitrary")),
    )(m, u, deg, b2, p2)
    return y, s


def _mm_kernel(a_ref, c_ref, o_ref, acc_ref, *, nk, bm, bn, add_eye,
               split_rhs):
    i, j, k = pl.program_id(0), pl.program_id(1), pl.program_id(2)

    @pl.when(k == 0)
    def _():
        acc_ref[...] = jnp.zeros_like(acc_ref)

    if split_rhs:
        acc_ref[...] += _split_dot(a_ref[...], c_ref[...])
    else:
        acc_ref[...] += jnp.dot(a_ref[...], c_ref[...],
                                preferred_element_type=jnp.float32)

    @pl.when(k == nk - 1)
    def _():
        r = acc_ref[...]
        if add_eye:
            gi = i * bm + jax.lax.broadcasted_iota(jnp.int32, (bm, bn), 0)
            gj = j * bn + jax.lax.broadcasted_iota(jnp.int32, (bm, bn), 1)
            r = r + jnp.where(gi == gj, 1.0, 0.0)
        o_ref[...] = r.astype(o_ref.dtype)


def _matmul(a, c, add_eye=False, out_dtype=jnp.float32, split_rhs=False,
            bm=512, bn=512, bk=1024):
    m, kk = a.shape
    n = c.shape[1]
    bk = min(bk, kk)
    bn = min(bn, n)
    nk = kk // bk
    return pl.pallas_call(
        functools.partial(_mm_kernel, nk=nk, bm=bm, bn=bn, add_eye=add_eye,
                          split_rhs=split_rhs),
        grid=(m // bm, n // bn, nk),
        in_specs=[
            pl.BlockSpec((bm, bk), lambda i, j, k: (i, k)),
            pl.BlockSpec((bk, bn), lambda i, j, k: (k, j)),
        ],
        out_specs=pl.BlockSpec((bm, bn), lambda i, j, k: (i, j)),
        out_shape=jax.ShapeDtypeStruct((m, n), out_dtype),
        scratch_shapes=[pltpu.VMEM((bm, bn), jnp.float32)],
        compiler_params=pltpu.CompilerParams(
            dimension_semantics=("parallel", "parallel", "arbitrary")),
    )(a, c)


def kernel(x, edge_index, W0, b0, W1, b1, W2, b2, W3, b3, W4, b4, p0, p1):
    n = x.shape[0]
    idx = jnp.arange(n, dtype=edge_index.dtype)
    rows = jnp.concatenate([edge_index[1], idx])
    cols = jnp.concatenate([edge_index[0], idx])
    m0 = (jnp.zeros((n, n), jnp.float32).at[rows, cols].add(1.0)
          .astype(_BF))
    deg0 = _rowsum(m0)

    u0 = _xw(x, W0, deg0)
    y0, s0 = _conv_apply(m0, u0, deg0, b0, p0)
    s0 = s0[:, 0]
    k0 = n // 2
    _, perm0 = jax.lax.top_k(s0, k0)
    x1 = y0[perm0] * jnp.tanh(s0[perm0])[:, None]
    m0q = m0[:, perm0]
    m1 = _matmul(m0[perm0, :], m0q, add_eye=True, out_dtype=_BF)
    deg1 = _rowsum(m1)

    u1 = _xw(x1, W1, deg1)
    y1, s1 = _conv_apply(m1, u1, deg1, b1, p1)
    s1 = s1[:, 0]
    k1 = k0 // 2
    _, perm1 = jax.lax.top_k(s1, k1)
    x2 = y1[perm1] * jnp.tanh(s1[perm1])[:, None]
    m1q = m1[:, perm1]
    m2 = _matmul(m1[perm1, :], m1q, add_eye=True)
    deg2 = _rowsum(m2)

    u2 = _xw(x2, W2, deg2)
    y2 = _conv_apply(m2, u2, deg2, b2, bk=512)

    u3b = _xw(y2, W3, deg1[perm1])
    init3 = _matmul(m1q, u3b, split_rhs=True)
    u3a = _xw(y1, W3, deg1)
    x3 = _conv_apply(m1, u3a, deg1, b3, init=init3)

    u4b = _xw(x3, W4, deg0[perm0])
    init4 = _matmul(m0q, u4b, split_rhs=True)
    u4a = _xw(y0, W4, deg0)
    x4 = _conv_apply(m0, u4a, deg0, b4, init=init4)
    return x4


# speedup vs baseline: 1.2441x; 1.2058x over previous
import functools

import jax
import jax.numpy as jnp
from jax import lax
from jax.experimental import pallas as pl
from jax.experimental.pallas import tpu as pltpu
from jax.experimental.pallas import tpu_sc as plsc

_BF = jnp.bfloat16


_N = 4096
_EDGES = 135168
_EPT = _EDGES // 16
_NCH = _EPT // 128
_RROWS = 256
_RWORDS = _RROWS * _N
_DUMPW = 1024
_ZPT = (_RWORDS + _DUMPW) // 16
_ZB = _ZPT // 8


def _sc_build_body(rows_hbm, cols_hbm, out_hbm, ed_v, es_v, idx_v, ones_v,
                   zeros_v, region):
    c = lax.axis_index("c")
    t = lax.axis_index("s")

    pltpu.sync_copy(rows_hbm.at[pl.ds(t * _EPT, _EPT)], ed_v)
    pltpu.sync_copy(cols_hbm.at[pl.ds(t * _EPT, _EPT)], es_v)

    def _fill_ones(i, carry):
        ones_v[pl.ds(i * 16, 16)] = jnp.full((16,), 1.0, jnp.float32)
        return carry

    lax.fori_loop(0, 128 // 16, _fill_ones, 0)

    def _fill_zeros(i, carry):
        zeros_v[pl.ds(i * 16, 16)] = jnp.full((16,), 0.0, jnp.float32)
        return carry

    lax.fori_loop(0, _ZB // 16, _fill_zeros, 0)

    def _pass(p, carry):
        lo = c * (_N // 2) + p * _RROWS

        def _zero(q, carry2):
            pltpu.sync_copy(zeros_v,
                            region.at[pl.ds(t * _ZPT + q * _ZB, _ZB)])
            return carry2

        lax.fori_loop(0, 8, _zero, 0)
        plsc.subcore_barrier()

        def _mkidx(i, carry2):
            d = ed_v[pl.ds(i * 16, 16)]
            s = es_v[pl.ds(i * 16, 16)]
            inr = (d >= lo) & (d < lo + _RROWS)
            flat = jnp.where(inr, (d - lo) * _N + s,
                             _RWORDS + (s & (_DUMPW - 1)))
            idx_v[i // 8, pl.ds((i % 8) * 16, 16)] = flat
            return carry2

        lax.fori_loop(0, _EPT // 16, _mkidx, 0)

        def _scat(j, carry2):
            pltpu.sync_copy(ones_v, region.at[idx_v.at[j]], add=True)
            return carry2

        lax.fori_loop(0, _NCH, _scat, 0)
        plsc.subcore_barrier()
        base = lo * _N + t * (_RWORDS // 16)
        pltpu.sync_copy(region.at[pl.ds(t * (_RWORDS // 16), _RWORDS // 16)],
                        out_hbm.at[pl.ds(base, _RWORDS // 16)])
        plsc.subcore_barrier()
        return carry

    lax.fori_loop(0, (_N // 2) // _RROWS, _pass, 0)


def _sc_build(rows, cols):
    mesh = plsc.VectorSubcoreMesh(core_axis_name="c", subcore_axis_name="s")
    f = functools.partial(
        pl.kernel,
        mesh=mesh,
        out_type=jax.ShapeDtypeStruct((_N * _N,), jnp.float32),
        scratch_types=[
            pltpu.VMEM((_EPT,), jnp.int32),
            pltpu.VMEM((_EPT,), jnp.int32),
            pltpu.VMEM((_NCH, 128), jnp.int32),
            pltpu.VMEM((128,), jnp.float32),
            pltpu.VMEM((_ZB,), jnp.float32),
            pltpu.VMEM_SHARED((_RWORDS + _DUMPW,), jnp.float32),
        ],
    )(_sc_build_body)
    return f(rows, cols)


def _split_dot(m_bf, u_f32, acc_dtype=jnp.float32):
    uh = u_f32.astype(_BF)
    ul = (u_f32 - uh.astype(jnp.float32)).astype(_BF)
    return (jnp.dot(m_bf, uh, preferred_element_type=acc_dtype)
            + jnp.dot(m_bf, ul, preferred_element_type=acc_dtype))


def _rowsum_kernel(m_ref, o_ref):
    o_ref[...] = jnp.sum(m_ref[...].astype(jnp.float32), axis=1,
                         keepdims=True)


def _rowsum(m, bm=512):
    n = m.shape[0]
    return pl.pallas_call(
        _rowsum_kernel,
        grid=(n // bm,),
        in_specs=[pl.BlockSpec((bm, m.shape[1]), lambda i: (i, 0))],
        out_specs=pl.BlockSpec((bm, 1), lambda i: (i, 0)),
        out_shape=jax.ShapeDtypeStruct((n, 1), jnp.float32),
    )(m)


def _xw_kernel(x_ref, w_ref, deg_ref, o_ref):
    dinv = jax.lax.rsqrt(deg_ref[...])
    o_ref[...] = dinv * jnp.dot(x_ref[...], w_ref[...],
                                preferred_element_type=jnp.float32)


def _xw(x, w, deg, bm=1024):
    n, cin = x.shape
    bm = min(bm, n)
    cout = w.shape[1]
    return pl.pallas_call(
        _xw_kernel,
        grid=(n // bm,),
        in_specs=[
            pl.BlockSpec((bm, cin), lambda i: (i, 0)),
            pl.BlockSpec((cin, cout), lambda i: (0, 0)),
            pl.BlockSpec((bm, 1), lambda i: (i, 0)),
        ],
        out_specs=pl.BlockSpec((bm, cout), lambda i: (i, 0)),
        out_shape=jax.ShapeDtypeStruct((n, cout), jnp.float32),
    )(x, w, deg)


def _spmv_kernel(m_ref, u_ref, deg_ref, b_ref, init_ref, o_ref, acc_ref, *,
                 nk, has_init, split):
    k = pl.program_id(1)

    @pl.when(k == 0)
    def _():
        if has_init:
            acc_ref[...] = init_ref[...]
        else:
            acc_ref[...] = jnp.zeros_like(acc_ref)

    if split:
        acc_ref[...] += _split_dot(m_ref[...].astype(_BF), u_ref[...])
    else:
        acc_ref[...] += jnp.dot(m_ref[...], u_ref[...],
                                preferred_element_type=jnp.float32)

    @pl.when(k == nk - 1)
    def _():
        dinv = jax.lax.rsqrt(deg_ref[...])
        o_ref[...] = jnp.maximum(dinv * acc_ref[...] + b_ref[...], 0.0)


def _spmv_score_kernel(m_ref, u_ref, deg_ref, b_ref, p_ref, o_ref, s_ref,
                       acc_ref, *, nk, split):
    k = pl.program_id(1)

    @pl.when(k == 0)
    def _():
        acc_ref[...] = jnp.zeros_like(acc_ref)

    if split:
        acc_ref[...] += _split_dot(m_ref[...].astype(_BF), u_ref[...])
    else:
        acc_ref[...] += jnp.dot(m_ref[...], u_ref[...],
                                preferred_element_type=jnp.float32)

    @pl.when(k == nk - 1)
    def _():
        dinv = jax.lax.rsqrt(deg_ref[...])
        y = jnp.maximum(dinv * acc_ref[...] + b_ref[...], 0.0)
        o_ref[...] = y
        p = p_ref[...]
        pn = p / jnp.sqrt(jnp.sum(p * p))
        s_ref[...] = jnp.dot(y, pn.T, preferred_element_type=jnp.float32)


def _conv_apply(m, u, deg, b, p=None, init=None, bm=512, bk=1024, split=True):
    n = m.shape[0]
    cout = u.shape[1]
    nk = m.shape[1] // bk
    b2 = b.reshape(1, cout)
    if p is None:
        has_init = init is not None
        if init is None:
            init = jnp.zeros((1, cout), jnp.float32)
            init_spec = pl.BlockSpec((1, cout), lambda i, k: (0, 0))
        else:
            init_spec = pl.BlockSpec((bm, cout), lambda i, k: (i, 0))
        return pl.pallas_call(
            functools.partial(_spmv_kernel, nk=nk, has_init=has_init,
                              split=split),
            grid=(n // bm, nk),
            in_specs=[
                pl.BlockSpec((bm, bk), lambda i, k: (i, k)),
                pl.BlockSpec((bk, cout), lambda i, k: (k, 0)),
                pl.BlockSpec((bm, 1), lambda i, k: (i, 0)),
                pl.BlockSpec((1, cout), lambda i, k: (0, 0)),
                init_spec,
            ],
            out_specs=pl.BlockSpec((bm, cout), lambda i, k: (i, 0)),
            out_shape=jax.ShapeDtypeStruct((n, cout), jnp.float32),
            scratch_shapes=[pltpu.VMEM((bm, cout), jnp.float32)],
            compiler_params=pltpu.CompilerParams(
                dimension_semantics=("parallel", "arbitrary")),
        )(m, u, deg, b2, init)
    p2 = p.reshape(1, cout)
    y, s = pl.pallas_call(
        functools.partial(_spmv_score_kernel, nk=nk, split=split),
        grid=(n // bm, nk),
        in_specs=[
            pl.BlockSpec((bm, bk), lambda i, k: (i, k)),
            pl.BlockSpec((bk, cout), lambda i, k: (k, 0)),
            pl.BlockSpec((bm, 1), lambda i, k: (i, 0)),
            pl.BlockSpec((1, cout), lambda i, k: (0, 0)),
            pl.BlockSpec((1, cout), lambda i, k: (0, 0)),
        ],
        out_specs=[
            pl.BlockSpec((bm, cout), lambda i, k: (i, 0)),
            pl.BlockSpec((bm, 1), lambda i, k: (i, 0)),
        ],
        out_shape=[
            jax.ShapeDtypeStruct((n, cout), jnp.float32),
            jax.ShapeDtypeStruct((n, 1), jnp.float32),
        ],
        scratch_shapes=[pltpu.VMEM((bm, cout), jnp.float32)],
        compiler_params=pltpu.CompilerParams(
            dimension_semantics=("parallel", "arbitrary")),
    )(m, u, deg, b2, p2)
    return y, s


def _mm_kernel(a_ref, c_ref, o_ref, acc_ref, *, nk, bm, bn, add_eye, mode):
    i, j, k = pl.program_id(0), pl.program_id(1), pl.program_id(2)

    @pl.when(k == 0)
    def _():
        acc_ref[...] = jnp.zeros_like(acc_ref)

    if mode == "bf16split":
        acc_ref[...] += _split_dot(a_ref[...].astype(_BF), c_ref[...])
    elif mode == "bf16":
        acc_ref[...] += jnp.dot(a_ref[...].astype(_BF), c_ref[...].astype(_BF),
                                preferred_element_type=jnp.float32)
    else:
        acc_ref[...] += jnp.dot(a_ref[...], c_ref[...],
                                preferred_element_type=jnp.float32)

    @pl.when(k == nk - 1)
    def _():
        r = acc_ref[...]
        if add_eye:
            gi = i * bm + jax.lax.broadcasted_iota(jnp.int32, (bm, bn), 0)
            gj = j * bn + jax.lax.broadcasted_iota(jnp.int32, (bm, bn), 1)
            r = r + jnp.where(gi == gj, 1.0, 0.0)
        o_ref[...] = r.astype(o_ref.dtype)


def _matmul(a, c, add_eye=False, out_dtype=jnp.float32, mode="bf16",
            bm=512, bn=512, bk=1024):
    m, kk = a.shape
    n = c.shape[1]
    bk = min(bk, kk)
    bn = min(bn, n)
    nk = kk // bk
    return pl.pallas_call(
        functools.partial(_mm_kernel, nk=nk, bm=bm, bn=bn, add_eye=add_eye,
                          mode=mode),
        grid=(m // bm, n // bn, nk),
        in_specs=[
            pl.BlockSpec((bm, bk), lambda i, j, k: (i, k)),
            pl.BlockSpec((bk, bn), lambda i, j, k: (k, j)),
        ],
        out_specs=pl.BlockSpec((bm, bn), lambda i, j, k: (i, j)),
        out_shape=jax.ShapeDtypeStruct((m, n), out_dtype),
        scratch_shapes=[pltpu.VMEM((bm, bn), jnp.float32)],
        compiler_params=pltpu.CompilerParams(
            dimension_semantics=("parallel", "parallel", "arbitrary")),
    )(a, c)


def kernel(x, edge_index, W0, b0, W1, b1, W2, b2, W3, b3, W4, b4, p0, p1):
    n = x.shape[0]
    idx = jnp.arange(n, dtype=edge_index.dtype)
    rows = jnp.concatenate([edge_index[1], idx])
    cols = jnp.concatenate([edge_index[0], idx])
    m0 = _sc_build(rows, cols).reshape(n, n)
    deg0 = _rowsum(m0)

    u0 = _xw(x, W0, deg0)
    y0, s0 = _conv_apply(m0, u0, deg0, b0, p0)
    s0 = s0[:, 0]
    k0 = n // 2
    _, perm0 = jax.lax.top_k(s0, k0)
    x1 = y0[perm0] * jnp.tanh(s0[perm0])[:, None]
    m0q = m0[:, perm0]
    m1 = _matmul(m0[perm0, :], m0q, add_eye=True, mode="bf16")
    deg1 = _rowsum(m1)

    u1 = _xw(x1, W1, deg1)
    y1, s1 = _conv_apply(m1, u1, deg1, b1, p1)
    s1 = s1[:, 0]
    k1 = k0 // 2
    _, perm1 = jax.lax.top_k(s1, k1)
    x2 = y1[perm1] * jnp.tanh(s1[perm1])[:, None]
    m1q = m1[:, perm1]
    m2 = _matmul(m1[perm1, :], m1q, add_eye=True, mode="bf16")
    deg2 = _rowsum(m2)

    u2 = _xw(x2, W2, deg2)
    y2 = _conv_apply(m2, u2, deg2, b2, bk=512, split=False)

    u3b = _xw(y2, W3, deg1[perm1])
    init3 = _matmul(m1q, u3b, mode="bf16split")
    u3a = _xw(y1, W3, deg1)
    x3 = _conv_apply(m1, u3a, deg1, b3, init=init3)

    u4b = _xw(x3, W4, deg0[perm0])
    init4 = _matmul(m0q, u4b, mode="bf16split")
    u4a = _xw(y0, W4, deg0)
    x4 = _conv_apply(m0, u4a, deg0, b4, init=init4)
    return x4


# speedup vs baseline: 1.2833x; 1.0315x over previous
import functools

import jax
import jax.numpy as jnp
from jax import lax
from jax.experimental import pallas as pl
from jax.experimental.pallas import tpu as pltpu
from jax.experimental.pallas import tpu_sc as plsc

_BF = jnp.bfloat16


_N = 4096
_EDGES = 135168
_EPT = _EDGES // 16
_NCH = _EPT // 128
_RROWS = 256
_RWORDS = _RROWS * _N
_DUMPW = 1024
_ZPT = (_RWORDS + _DUMPW) // 16
_ZB = _ZPT // 8


def _sc_build_body(rows_hbm, cols_hbm, out_hbm, ed_v, es_v, idx_v, ones_v,
                   zeros_v, region):
    c = lax.axis_index("c")
    t = lax.axis_index("s")

    pltpu.sync_copy(rows_hbm.at[pl.ds(t * _EPT, _EPT)], ed_v)
    pltpu.sync_copy(cols_hbm.at[pl.ds(t * _EPT, _EPT)], es_v)

    def _fill_ones(i, carry):
        ones_v[pl.ds(i * 16, 16)] = jnp.full((16,), 1.0, jnp.float32)
        return carry

    lax.fori_loop(0, 128 // 16, _fill_ones, 0)

    def _fill_zeros(i, carry):
        zeros_v[pl.ds(i * 16, 16)] = jnp.full((16,), 0.0, jnp.float32)
        return carry

    lax.fori_loop(0, _ZB // 16, _fill_zeros, 0)

    def _pass(p, carry):
        lo = c * (_N // 2) + p * _RROWS

        def _zero(q, carry2):
            pltpu.sync_copy(zeros_v,
                            region.at[pl.ds(t * _ZPT + q * _ZB, _ZB)])
            return carry2

        lax.fori_loop(0, 8, _zero, 0)
        plsc.subcore_barrier()

        def _mkidx(i, carry2):
            d = ed_v[pl.ds(i * 16, 16)]
            s = es_v[pl.ds(i * 16, 16)]
            inr = (d >= lo) & (d < lo + _RROWS)
            flat = jnp.where(inr, (d - lo) * _N + s,
                             _RWORDS + (s & (_DUMPW - 1)))
            idx_v[i // 8, pl.ds((i % 8) * 16, 16)] = flat
            return carry2

        lax.fori_loop(0, _EPT // 16, _mkidx, 0)

        def _scat(j, carry2):
            pltpu.sync_copy(ones_v, region.at[idx_v.at[j]], add=True)
            return carry2

        lax.fori_loop(0, _NCH, _scat, 0)
        plsc.subcore_barrier()
        base = lo * _N + t * (_RWORDS // 16)
        pltpu.sync_copy(region.at[pl.ds(t * (_RWORDS // 16), _RWORDS // 16)],
                        out_hbm.at[pl.ds(base, _RWORDS // 16)])
        plsc.subcore_barrier()
        return carry

    lax.fori_loop(0, (_N // 2) // _RROWS, _pass, 0)


def _sc_build(rows, cols):
    mesh = plsc.VectorSubcoreMesh(core_axis_name="c", subcore_axis_name="s")
    f = functools.partial(
        pl.kernel,
        mesh=mesh,
        out_type=jax.ShapeDtypeStruct((_N * _N,), jnp.float32),
        scratch_types=[
            pltpu.VMEM((_EPT,), jnp.int32),
            pltpu.VMEM((_EPT,), jnp.int32),
            pltpu.VMEM((_NCH, 128), jnp.int32),
            pltpu.VMEM((128,), jnp.float32),
            pltpu.VMEM((_ZB,), jnp.float32),
            pltpu.VMEM_SHARED((_RWORDS + _DUMPW,), jnp.float32),
        ],
    )(_sc_build_body)
    return f(rows, cols)


def _split_dot(m_bf, u_f32, acc_dtype=jnp.float32):
    uh = u_f32.astype(_BF)
    ul = (u_f32 - uh.astype(jnp.float32)).astype(_BF)
    return (jnp.dot(m_bf, uh, preferred_element_type=acc_dtype)
            + jnp.dot(m_bf, ul, preferred_element_type=acc_dtype))


def _cast2d_kernel(x_ref, o_ref, *, bm, n):
    o_ref[...] = x_ref[...].reshape(bm, n).astype(_BF)


def _cast2d(x_flat, n, bm=128):
    return pl.pallas_call(
        functools.partial(_cast2d_kernel, bm=bm, n=n),
        grid=(n // bm,),
        in_specs=[pl.BlockSpec((bm * n,), lambda i: (i,))],
        out_specs=pl.BlockSpec((bm, n), lambda i: (i, 0)),
        out_shape=jax.ShapeDtypeStruct((n, n), _BF),
    )(x_flat)


def _rowsum_kernel(m_ref, o_ref):
    o_ref[...] = jnp.sum(m_ref[...].astype(jnp.float32), axis=1,
                         keepdims=True)


def _rowsum(m, bm=512):
    n = m.shape[0]
    return pl.pallas_call(
        _rowsum_kernel,
        grid=(n // bm,),
        in_specs=[pl.BlockSpec((bm, m.shape[1]), lambda i: (i, 0))],
        out_specs=pl.BlockSpec((bm, 1), lambda i: (i, 0)),
        out_shape=jax.ShapeDtypeStruct((n, 1), jnp.float32),
    )(m)


def _xw_kernel(x_ref, w_ref, deg_ref, o_ref):
    dinv = jax.lax.rsqrt(deg_ref[...])
    o_ref[...] = dinv * jnp.dot(x_ref[...], w_ref[...],
                                preferred_element_type=jnp.float32)


def _xw(x, w, deg, bm=1024):
    n, cin = x.shape
    bm = min(bm, n)
    cout = w.shape[1]
    return pl.pallas_call(
        _xw_kernel,
        grid=(n // bm,),
        in_specs=[
            pl.BlockSpec((bm, cin), lambda i: (i, 0)),
            pl.BlockSpec((cin, cout), lambda i: (0, 0)),
            pl.BlockSpec((bm, 1), lambda i: (i, 0)),
        ],
        out_specs=pl.BlockSpec((bm, cout), lambda i: (i, 0)),
        out_shape=jax.ShapeDtypeStruct((n, cout), jnp.float32),
    )(x, w, deg)


def _spmv_kernel(m_ref, u_ref, deg_ref, b_ref, init_ref, o_ref, acc_ref, *,
                 nk, has_init, split):
    k = pl.program_id(1)

    @pl.when(k == 0)
    def _():
        if has_init:
            acc_ref[...] = init_ref[...]
        else:
            acc_ref[...] = jnp.zeros_like(acc_ref)

    if split:
        acc_ref[...] += _split_dot(m_ref[...].astype(_BF), u_ref[...])
    else:
        acc_ref[...] += jnp.dot(m_ref[...], u_ref[...],
                                preferred_element_type=jnp.float32)

    @pl.when(k == nk - 1)
    def _():
        dinv = jax.lax.rsqrt(deg_ref[...])
        o_ref[...] = jnp.maximum(dinv * acc_ref[...] + b_ref[...], 0.0)


def _spmv_score_kernel(m_ref, u_ref, deg_ref, b_ref, p_ref, o_ref, s_ref,
                       acc_ref, *, nk, split):
    k = pl.program_id(1)

    @pl.when(k == 0)
    def _():
        acc_ref[...] = jnp.zeros_like(acc_ref)

    if split:
        acc_ref[...] += _split_dot(m_ref[...].astype(_BF), u_ref[...])
    else:
        acc_ref[...] += jnp.dot(m_ref[...], u_ref[...],
                                preferred_element_type=jnp.float32)

    @pl.when(k == nk - 1)
    def _():
        dinv = jax.lax.rsqrt(deg_ref[...])
        y = jnp.maximum(dinv * acc_ref[...] + b_ref[...], 0.0)
        o_ref[...] = y
        p = p_ref[...]
        pn = p / jnp.sqrt(jnp.sum(p * p))
        s_ref[...] = jnp.dot(y, pn.T, preferred_element_type=jnp.float32)


def _conv_apply(m, u, deg, b, p=None, init=None, bm=512, bk=1024, split=True):
    n = m.shape[0]
    cout = u.shape[1]
    nk = m.shape[1] // bk
    b2 = b.reshape(1, cout)
    if p is None:
        has_init = init is not None
        if init is None:
            init = jnp.zeros((1, cout), jnp.float32)
            init_spec = pl.BlockSpec((1, cout), lambda i, k: (0, 0))
        else:
            init_spec = pl.BlockSpec((bm, cout), lambda i, k: (i, 0))
        return pl.pallas_call(
            functools.partial(_spmv_kernel, nk=nk, has_init=has_init,
                              split=split),
            grid=(n // bm, nk),
            in_specs=[
                pl.BlockSpec((bm, bk), lambda i, k: (i, k)),
                pl.BlockSpec((bk, cout), lambda i, k: (k, 0)),
                pl.BlockSpec((bm, 1), lambda i, k: (i, 0)),
                pl.BlockSpec((1, cout), lambda i, k: (0, 0)),
                init_spec,
            ],
            out_specs=pl.BlockSpec((bm, cout), lambda i, k: (i, 0)),
            out_shape=jax.ShapeDtypeStruct((n, cout), jnp.float32),
            scratch_shapes=[pltpu.VMEM((bm, cout), jnp.float32)],
            compiler_params=pltpu.CompilerParams(
                dimension_semantics=("parallel", "arbitrary")),
        )(m, u, deg, b2, init)
    p2 = p.reshape(1, cout)
    y, s = pl.pallas_call(
        functools.partial(_spmv_score_kernel, nk=nk, split=split),
        grid=(n // bm, nk),
        in_specs=[
            pl.BlockSpec((bm, bk), lambda i, k: (i, k)),
            pl.BlockSpec((bk, cout), lambda i, k: (k, 0)),
            pl.BlockSpec((bm, 1), lambda i, k: (i, 0)),
            pl.BlockSpec((1, cout), lambda i, k: (0, 0)),
            pl.BlockSpec((1, cout), lambda i, k: (0, 0)),
        ],
        out_specs=[
            pl.BlockSpec((bm, cout), lambda i, k: (i, 0)),
            pl.BlockSpec((bm, 1), lambda i, k: (i, 0)),
        ],
        out_shape=[
            jax.ShapeDtypeStruct((n, cout), jnp.float32),
            jax.ShapeDtypeStruct((n, 1), jnp.float32),
        ],
        scratch_shapes=[pltpu.VMEM((bm, cout), jnp.float32)],
        compiler_params=pltpu.CompilerParams(
            dimension_semantics=("parallel", "arbitrary")),
    )(m, u, deg, b2, p2)
    return y, s


def _mm_kernel(a_ref, c_ref, o_ref, acc_ref, *, nk, bm, bn, add_eye, mode):
    i, j, k = pl.program_id(0), pl.program_id(1), pl.program_id(2)

    @pl.when(k == 0)
    def _():
        acc_ref[...] = jnp.zeros_like(acc_ref)

    if mode == "bf16split":
        acc_ref[...] += _split_dot(a_ref[...].astype(_BF), c_ref[...])
    elif mode == "bf16":
        acc_ref[...] += jnp.dot(a_ref[...].astype(_BF), c_ref[...].astype(_BF),
                                preferred_element_type=jnp.float32)
    else:
        acc_ref[...] += jnp.dot(a_ref[...], c_ref[...],
                                preferred_element_type=jnp.float32)

    @pl.when(k == nk - 1)
    def _():
        r = acc_ref[...]
        if add_eye:
            gi = i * bm + jax.lax.broadcasted_iota(jnp.int32, (bm, bn), 0)
            gj = j * bn + jax.lax.broadcasted_iota(jnp.int32, (bm, bn), 1)
            r = r + jnp.where(gi == gj, 1.0, 0.0)
        o_ref[...] = r.astype(o_ref.dtype)


def _matmul(a, c, add_eye=False, out_dtype=jnp.float32, mode="bf16",
            bm=512, bn=512, bk=1024):
    m, kk = a.shape
    n = c.shape[1]
    bk = min(bk, kk)
    bn = min(bn, n)
    nk = kk // bk
    return pl.pallas_call(
        functools.partial(_mm_kernel, nk=nk, bm=bm, bn=bn, add_eye=add_eye,
                          mode=mode),
        grid=(m // bm, n // bn, nk),
        in_specs=[
            pl.BlockSpec((bm, bk), lambda i, j, k: (i, k)),
            pl.BlockSpec((bk, bn), lambda i, j, k: (k, j)),
        ],
        out_specs=pl.BlockSpec((bm, bn), lambda i, j, k: (i, j)),
        out_shape=jax.ShapeDtypeStruct((m, n), out_dtype),
        scratch_shapes=[pltpu.VMEM((bm, bn), jnp.float32)],
        compiler_params=pltpu.CompilerParams(
            dimension_semantics=("parallel", "parallel", "arbitrary")),
    )(a, c)


def kernel(x, edge_index, W0, b0, W1, b1, W2, b2, W3, b3, W4, b4, p0, p1):
    n = x.shape[0]
    idx = jnp.arange(n, dtype=edge_index.dtype)
    rows = jnp.concatenate([edge_index[1], idx])
    cols = jnp.concatenate([edge_index[0], idx])
    m0 = _cast2d(_sc_build(rows, cols), n)
    deg0 = _rowsum(m0)

    u0 = _xw(x, W0, deg0)
    y0, s0 = _conv_apply(m0, u0, deg0, b0, p0)
    s0 = s0[:, 0]
    k0 = n // 2
    _, perm0 = jax.lax.top_k(s0, k0)
    x1 = y0[perm0] * jnp.tanh(s0[perm0])[:, None]
    m0q = m0[:, perm0]
    m1 = _matmul(m0[perm0, :], m0q, add_eye=True, mode="bf16",
                 out_dtype=_BF)
    deg1 = _rowsum(m1)

    u1 = _xw(x1, W1, deg1)
    y1, s1 = _conv_apply(m1, u1, deg1, b1, p1)
    s1 = s1[:, 0]
    k1 = k0 // 2
    _, perm1 = jax.lax.top_k(s1, k1)
    x2 = y1[perm1] * jnp.tanh(s1[perm1])[:, None]
    m1q = m1[:, perm1]
    m2 = _matmul(m1[perm1, :], m1q, add_eye=True, mode="bf16")
    deg2 = _rowsum(m2)

    u2 = _xw(x2, W2, deg2)
    y2 = _conv_apply(m2, u2, deg2, b2, bk=512, split=False)

    u3b = _xw(y2, W3, deg1[perm1])
    init3 = _matmul(m1q, u3b, mode="bf16split")
    u3a = _xw(y1, W3, deg1)
    x3 = _conv_apply(m1, u3a, deg1, b3, init=init3)

    u4b = _xw(x3, W4, deg0[perm0])
    init4 = _matmul(m0q, u4b, mode="bf16split")
    u4a = _xw(y0, W4, deg0)
    x4 = _conv_apply(m0, u4a, deg0, b4, init=init4)
    return x4
